# baseline (speedup 1.0000x reference)
import functools

import jax
import jax.numpy as jnp
from jax import lax
from jax.experimental import pallas as pl
from jax.experimental.pallas import tpu as pltpu
from jax.experimental.pallas import tpu_sc as plsc

BATCH = 4096
SEQ = 200
DIM = 64

NC = 2
NS = 16
NW = NC * NS
ROWS_PER_W = BATCH // NW
G = 2
NG = ROWS_PER_W // G
GIDX = G * SEQ
IDX_PER_W = ROWS_PER_W * SEQ
INV_SEQ = 1.0 / SEQ


def _build_kernel():
    mesh = plsc.VectorSubcoreMesh(core_axis_name="c", subcore_axis_name="s")

    @functools.partial(
        pl.kernel,
        out_type=jax.ShapeDtypeStruct((BATCH, DIM), jnp.float32),
        mesh=mesh,
        compiler_params=pltpu.CompilerParams(
            use_tc_tiling_on_sc=False, needs_layout_passes=False
        ),
        scratch_types=[
            pltpu.VMEM((IDX_PER_W,), jnp.int32),
            pltpu.VMEM((2, GIDX, DIM), jnp.bfloat16),
            pltpu.VMEM((ROWS_PER_W, DIM), jnp.float32),
            pltpu.SemaphoreType.DMA,
            pltpu.SemaphoreType.DMA,
        ],
    )
    def enc(ids_hbm, table_hbm, out_hbm, idx_v, rows_v, out_v, sem0, sem1):
        sems = (sem0, sem1)
        wid = lax.axis_index("s") * NC + lax.axis_index("c")

        pltpu.sync_copy(ids_hbm.at[pl.ds(wid * IDX_PER_W, IDX_PER_W)], idx_v)

        def fire(gg, b):
            base = gg * GIDX
            for r in range(G):
                for off, n in ((0, 128), (128, SEQ - 128)):
                    pltpu.async_copy(
                        table_hbm.at[idx_v.at[pl.ds(base + r * SEQ + off, n)]],
                        rows_v.at[b, pl.ds(r * SEQ + off, n)],
                        sems[b],
                    )

        def drain(b):
            pltpu.make_async_copy(
                table_hbm.at[pl.ds(0, GIDX)], rows_v.at[b], sems[b]
            ).wait()

        evens = lax.iota(jnp.int32, 16) * 2
        odds = evens + 1

        def accum(gg, b):
            for r in range(G):
                rbase = r * SEQ

                def body(j, accs, _rbase=rbase):
                    ae0, ao0, ae1, ao1 = accs
                    row = _rbase + j
                    v0 = rows_v[b, row, pl.ds(0, 32)]
                    v1 = rows_v[b, row, pl.ds(32, 32)]
                    e0, o0 = plsc.unpack(v0, format=plsc.PackFormat.INTERLEAVED)
                    e1, o1 = plsc.unpack(v1, format=plsc.PackFormat.INTERLEAVED)
                    return ae0 + e0, ao0 + o0, ae1 + e1, ao1 + o1

                z = jnp.zeros((16,), jnp.float32)
                ae0, ao0, ae1, ao1 = lax.fori_loop(0, SEQ, body, (z, z, z, z))
                orow = jnp.full((16,), gg * G + r, jnp.int32)
                plsc.store_scatter(out_v, [orow, evens], ae0 * INV_SEQ)
                plsc.store_scatter(out_v, [orow, odds], ao0 * INV_SEQ)
                plsc.store_scatter(out_v, [orow, evens + 32], ae1 * INV_SEQ)
                plsc.store_scatter(out_v, [orow, odds + 32], ao1 * INV_SEQ)

        fire(0, 0)

        def outer(i, carry):
            g = i * 2
            fire(g + 1, 1)
            drain(0)
            accum(g, 0)

            @pl.when(g + 2 < NG)
            def _():
                fire(g + 2, 0)

            drain(1)
            accum(g + 1, 1)
            return carry

        lax.fori_loop(0, NG // 2, outer, 0)

        pltpu.sync_copy(out_v, out_hbm.at[pl.ds(wid * ROWS_PER_W, ROWS_PER_W)])

    return enc


_enc = _build_kernel()


def kernel(text_ids, table):
    ids_flat = text_ids.reshape(-1).astype(jnp.int32)
    return _enc(ids_flat, table.astype(jnp.bfloat16))


# speedup vs baseline: 1.4236x; 1.4236x over previous
import functools

import jax
import jax.numpy as jnp
from jax import lax
from jax.experimental import pallas as pl
from jax.experimental.pallas import tpu as pltpu
from jax.experimental.pallas import tpu_sc as plsc

BATCH = 4096
SEQ = 200
DIM = 64
VOCAB_ROWS = 1000000

NC = 2
NS = 16
NW = NC * NS
ROWS_PER_W = BATCH // NW
G = 2
NG = ROWS_PER_W // G
GIDX = G * SEQ
IDX_PER_W = ROWS_PER_W * SEQ
INV_SEQ = 1.0 / SEQ


def _build_kernel():
    mesh = plsc.VectorSubcoreMesh(core_axis_name="c", subcore_axis_name="s")

    @functools.partial(
        pl.kernel,
        out_type=jax.ShapeDtypeStruct((BATCH, DIM), jnp.float32),
        mesh=mesh,
        compiler_params=pltpu.CompilerParams(
            use_tc_tiling_on_sc=False, needs_layout_passes=False
        ),
        scratch_types=[
            pltpu.VMEM((IDX_PER_W,), jnp.int32),
            pltpu.VMEM((2, GIDX, DIM), jnp.float32),
            pltpu.VMEM((ROWS_PER_W, DIM), jnp.float32),
            pltpu.SemaphoreType.DMA,
            pltpu.SemaphoreType.DMA,
        ],
    )
    def enc(ids_hbm, table_hbm, out_hbm, idx_v, rows_v, out_v, sem0, sem1):
        sems = (sem0, sem1)
        wid = lax.axis_index("s") * NC + lax.axis_index("c")

        pltpu.sync_copy(ids_hbm.at[pl.ds(wid * IDX_PER_W, IDX_PER_W)], idx_v)
        def dbl(k, carry):
            sl = pl.ds(k * 16, 16)
            idx_v[sl] = idx_v[sl] * 2
            return carry
        lax.fori_loop(0, IDX_PER_W // 16, dbl, 0)

        def fire(gg, b):
            base = gg * GIDX
            for r in range(G):
                for off, n in ((0, 128), (128, SEQ - 128)):
                    pltpu.async_copy(
                        table_hbm.at[idx_v.at[pl.ds(base + r * SEQ + off, n)]],
                        rows_v.at[b, pl.ds(r * SEQ + off, n)],
                        sems[b],
                    )

        def drain(b):
            pltpu.make_async_copy(
                table_hbm.at[pl.ds(0, GIDX)], rows_v.at[b], sems[b]
            ).wait()

        def accum(gg, b):
            for r in range(G):
                rbase = r * SEQ

                def body(j, accs, _rbase=rbase):
                    a0, a1, a2, a3 = accs
                    row = _rbase + j
                    a0 = a0 + rows_v[b, row, pl.ds(0, 16)]
                    a1 = a1 + rows_v[b, row, pl.ds(16, 16)]
                    a2 = a2 + rows_v[b, row, pl.ds(32, 16)]
                    a3 = a3 + rows_v[b, row, pl.ds(48, 16)]
                    return a0, a1, a2, a3

                z = jnp.zeros((16,), jnp.float32)
                a0, a1, a2, a3 = lax.fori_loop(0, SEQ, body, (z, z, z, z))
                orow = gg * G + r
                out_v[orow, pl.ds(0, 16)] = a0 * INV_SEQ
                out_v[orow, pl.ds(16, 16)] = a1 * INV_SEQ
                out_v[orow, pl.ds(32, 16)] = a2 * INV_SEQ
                out_v[orow, pl.ds(48, 16)] = a3 * INV_SEQ

        fire(0, 0)

        def outer(i, carry):
            g = i * 2
            fire(g + 1, 1)
            drain(0)
            accum(g, 0)

            @pl.when(g + 2 < NG)
            def _():
                fire(g + 2, 0)

            drain(1)
            accum(g + 1, 1)
            return carry

        lax.fori_loop(0, NG // 2, outer, 0)

        pltpu.sync_copy(out_v, out_hbm.at[pl.ds(wid * ROWS_PER_W, ROWS_PER_W)])

    return enc


_enc = _build_kernel()


def kernel(text_ids, table):
    ids_flat = text_ids.reshape(-1).astype(jnp.int32)
    table_p = jnp.pad(table, ((0, 0), (0, DIM))).reshape(2 * VOCAB_ROWS, DIM)
    return _enc(ids_flat, table_p)


# speedup vs baseline: 1.4555x; 1.0225x over previous
import functools

import jax
import jax.numpy as jnp
from jax import lax
from jax.experimental import pallas as pl
from jax.experimental.pallas import tpu as pltpu
from jax.experimental.pallas import tpu_sc as plsc

BATCH = 4096
SEQ = 200
DIM = 64
VOCAB_ROWS = 1000000

NC = 2
NS = 16
NW = NC * NS
ROWS_PER_W = BATCH // NW
G = 2
NG = ROWS_PER_W // G
GIDX = G * SEQ
IDX_PER_W = ROWS_PER_W * SEQ
INV_SEQ = 1.0 / SEQ


def _build_kernel():
    mesh = plsc.VectorSubcoreMesh(core_axis_name="c", subcore_axis_name="s")

    @functools.partial(
        pl.kernel,
        out_type=jax.ShapeDtypeStruct((BATCH, DIM), jnp.float32),
        mesh=mesh,
        compiler_params=pltpu.CompilerParams(
            use_tc_tiling_on_sc=False, needs_layout_passes=False
        ),
        scratch_types=[
            pltpu.VMEM((IDX_PER_W,), jnp.int32),
            pltpu.VMEM((2, GIDX, DIM), jnp.float32),
            pltpu.VMEM((ROWS_PER_W, DIM), jnp.float32),
            pltpu.SemaphoreType.DMA,
            pltpu.SemaphoreType.DMA,
        ],
    )
    def enc(ids_hbm, table_hbm, out_hbm, idx_v, rows_v, out_v, sem0, sem1):
        sems = (sem0, sem1)
        wid = lax.axis_index("s") * NC + lax.axis_index("c")

        pltpu.sync_copy(ids_hbm.at[pl.ds(wid * IDX_PER_W, IDX_PER_W)], idx_v)
        def dbl(k, carry):
            sl = pl.ds(k * 16, 16)
            idx_v[sl] = idx_v[sl] * 2
            return carry
        lax.fori_loop(0, IDX_PER_W // 16, dbl, 0)

        def fire(gg, b):
            base = gg * GIDX
            for r in range(G):
                for off, n in ((0, 128), (128, SEQ - 128)):
                    pltpu.async_copy(
                        table_hbm.at[idx_v.at[pl.ds(base + r * SEQ + off, n)]],
                        rows_v.at[b, pl.ds(r * SEQ + off, n)],
                        sems[b],
                    )

        def drain(b):
            pltpu.make_async_copy(
                table_hbm.at[pl.ds(0, GIDX)], rows_v.at[b], sems[b]
            ).wait()

        def accum(gg, b):
            for r in range(G):
                rbase = r * SEQ

                def body(j, accs, _rbase=rbase):
                    a0, a1, a2, a3 = accs
                    row = _rbase + j * 8
                    for u in range(8):
                        a0 = a0 + rows_v[b, row + u, pl.ds(0, 16)]
                        a1 = a1 + rows_v[b, row + u, pl.ds(16, 16)]
                        a2 = a2 + rows_v[b, row + u, pl.ds(32, 16)]
                        a3 = a3 + rows_v[b, row + u, pl.ds(48, 16)]
                    return a0, a1, a2, a3

                z = jnp.zeros((16,), jnp.float32)
                a0, a1, a2, a3 = lax.fori_loop(0, SEQ // 8, body, (z, z, z, z))
                orow = gg * G + r
                out_v[orow, pl.ds(0, 16)] = a0 * INV_SEQ
                out_v[orow, pl.ds(16, 16)] = a1 * INV_SEQ
                out_v[orow, pl.ds(32, 16)] = a2 * INV_SEQ
                out_v[orow, pl.ds(48, 16)] = a3 * INV_SEQ

        fire(0, 0)

        def outer(i, carry):
            g = i * 2
            fire(g + 1, 1)
            drain(0)
            accum(g, 0)

            @pl.when(g + 2 < NG)
            def _():
                fire(g + 2, 0)

            drain(1)
            accum(g + 1, 1)
            return carry

        lax.fori_loop(0, NG // 2, outer, 0)

        pltpu.sync_copy(out_v, out_hbm.at[pl.ds(wid * ROWS_PER_W, ROWS_PER_W)])

    return enc


_enc = _build_kernel()


def kernel(text_ids, table):
    ids_flat = text_ids.reshape(-1).astype(jnp.int32)
    table_p = jnp.pad(table, ((0, 0), (0, DIM))).reshape(2 * VOCAB_ROWS, DIM)
    return _enc(ids_flat, table_p)


# speedup vs baseline: 1.4704x; 1.0102x over previous
import functools

import jax
import jax.numpy as jnp
from jax import lax
from jax.experimental import pallas as pl
from jax.experimental.pallas import tpu as pltpu
from jax.experimental.pallas import tpu_sc as plsc

BATCH = 4096
SEQ = 200
DIM = 64
VOCAB_ROWS = 1000000

NC = 2
NS = 16
NW = NC * NS
ROWS_PER_W = BATCH // NW
G = 2
NG = ROWS_PER_W // G
GIDX = G * SEQ
IDX_PER_W = ROWS_PER_W * SEQ
INV_SEQ = 1.0 / SEQ


def _build_kernel():
    mesh = plsc.VectorSubcoreMesh(core_axis_name="c", subcore_axis_name="s")

    @functools.partial(
        pl.kernel,
        out_type=jax.ShapeDtypeStruct((BATCH, DIM), jnp.float32),
        mesh=mesh,
        compiler_params=pltpu.CompilerParams(
            use_tc_tiling_on_sc=False, needs_layout_passes=False
        ),
        scratch_types=[
            pltpu.VMEM((IDX_PER_W,), jnp.int32),
            pltpu.VMEM((2, GIDX, DIM), jnp.float32),
            pltpu.VMEM((ROWS_PER_W, DIM), jnp.float32),
            pltpu.SemaphoreType.DMA,
            pltpu.SemaphoreType.DMA,
        ],
    )
    def enc(ids_hbm, table_hbm, out_hbm, idx_v, rows_v, out_v, sem0, sem1):
        sems = (sem0, sem1)
        wid = lax.axis_index("s") * NC + lax.axis_index("c")

        pltpu.sync_copy(ids_hbm.at[pl.ds(wid * IDX_PER_W, IDX_PER_W)], idx_v)

        def fire(gg, b):
            base = gg * GIDX
            for r in range(G):
                for off, n in ((0, 128), (128, SEQ - 128)):
                    pltpu.async_copy(
                        table_hbm.at[idx_v.at[pl.ds(base + r * SEQ + off, n)]],
                        rows_v.at[b, pl.ds(r * SEQ + off, n)],
                        sems[b],
                    )

        def drain(b):
            pltpu.make_async_copy(
                table_hbm.at[pl.ds(0, GIDX)], rows_v.at[b], sems[b]
            ).wait()

        def accum(gg, b):
            for r in range(G):
                rbase = r * SEQ

                def body(j, accs, _rbase=rbase):
                    a0, a1, a2, a3 = accs
                    row = _rbase + j * 8
                    for u in range(8):
                        a0 = a0 + rows_v[b, row + u, pl.ds(0, 16)]
                        a1 = a1 + rows_v[b, row + u, pl.ds(16, 16)]
                        a2 = a2 + rows_v[b, row + u, pl.ds(32, 16)]
                        a3 = a3 + rows_v[b, row + u, pl.ds(48, 16)]
                    return a0, a1, a2, a3

                z = jnp.zeros((16,), jnp.float32)
                a0, a1, a2, a3 = lax.fori_loop(0, SEQ // 8, body, (z, z, z, z))
                orow = gg * G + r
                out_v[orow, pl.ds(0, 16)] = a0 * INV_SEQ
                out_v[orow, pl.ds(16, 16)] = a1 * INV_SEQ
                out_v[orow, pl.ds(32, 16)] = a2 * INV_SEQ
                out_v[orow, pl.ds(48, 16)] = a3 * INV_SEQ

        fire(0, 0)

        def outer(i, carry):
            g = i * 2
            fire(g + 1, 1)
            drain(0)
            accum(g, 0)

            @pl.when(g + 2 < NG)
            def _():
                fire(g + 2, 0)

            drain(1)
            accum(g + 1, 1)
            return carry

        lax.fori_loop(0, NG // 2, outer, 0)

        pltpu.sync_copy(out_v, out_hbm.at[pl.ds(wid * ROWS_PER_W, ROWS_PER_W)])

    return enc


_enc = _build_kernel()


def kernel(text_ids, table):
    ids_flat = text_ids.reshape(-1).astype(jnp.int32) * 2
    table_p = jnp.pad(table, ((0, 0), (0, DIM))).reshape(2 * VOCAB_ROWS, DIM)
    return _enc(ids_flat, table_p)
